# Initial kernel scaffold; baseline (speedup 1.0000x reference)
import jax
import jax.numpy as jnp
from jax.experimental import pallas as pl


def kernel(query, reference_points, W_off, b_off, W_attn, b_attn, W_val, b_val, W_out, b_out):
    raise NotImplementedError("write your pallas kernel here")


# speedup vs baseline: 40.8503x; 40.8503x over previous
import jax
import jax.numpy as jnp
from jax.experimental import pallas as pl
from jax.experimental.pallas import tpu as pltpu

NHEADS = 12
NPTS = 8
HDIM = 64
CHUNK = 256


def _fused_kernel(q_ref, rp_ref, wcat_ref, bcat_ref, wout_ref, bout_ref, out_ref):
    h = pl.program_id(1)
    L, E = q_ref.shape[1], q_ref.shape[2]
    q = q_ref[0]
    r = jnp.dot(q, wcat_ref[0], preferred_element_type=jnp.float32) + bcat_ref[0]
    val = r[:, 0:HDIM]
    xf = r[:, HDIM:HDIM + NPTS] + rp_ref[0, :, 0:1]
    yf = r[:, HDIM + NPTS:HDIM + 2 * NPTS] + rp_ref[0, :, 1:2]
    logits = r[:, HDIM + 2 * NPTS:HDIM + 3 * NPTS]
    m = jnp.max(logits, axis=1, keepdims=True)
    e = jnp.exp(logits - m)
    attn = e / jnp.sum(e, axis=1, keepdims=True)

    gw = 32
    sampled = jnp.zeros((L, HDIM), jnp.float32)
    for c in range(L // CHUNK):
        i = jax.lax.broadcasted_iota(jnp.int32, (1, CHUNK), 1)
        xg = (i % gw).astype(jnp.float32)
        yg = (i // gw + c * (CHUNK // gw)).astype(jnp.float32)
        acc = None
        for p in range(NPTS):
            ap = attn[:, p:p + 1]
            dx = jnp.abs(xg - xf[:, p:p + 1])
            txa = jnp.maximum(ap - ap * dx, 0.0)
            dy = jnp.abs(yg - yf[:, p:p + 1])
            ty = jnp.maximum(1.0 - dy, 0.0)
            term = txa * ty
            acc = term if acc is None else acc + term
        sampled = sampled + jnp.dot(acc, val[c * CHUNK:(c + 1) * CHUNK, :],
                                    preferred_element_type=jnp.float32)

    contrib = jnp.dot(sampled, wout_ref[0], preferred_element_type=jnp.float32)

    @pl.when(h == 0)
    def _():
        out_ref[0] = contrib + bout_ref[...]

    @pl.when(h != 0)
    def _():
        out_ref[0] = out_ref[0] + contrib


def kernel(query, reference_points, W_off, b_off, W_attn, b_attn, W_val, b_val, W_out, b_out):
    N, H, W, E = query.shape
    L = H * W
    qf = query.reshape(N, L, E)
    Wv = W_val.reshape(E, NHEADS, HDIM).transpose(1, 0, 2)
    Wo2 = W_off.reshape(E, NHEADS, NPTS, 2)
    Wox = float(W) * Wo2[..., 0].transpose(1, 0, 2)
    Woy = float(H) * Wo2[..., 1].transpose(1, 0, 2)
    Wa = W_attn.reshape(E, NHEADS, NPTS).transpose(1, 0, 2)
    Wcat = jnp.concatenate([Wv, Wox, Woy, Wa], axis=2)
    bo2 = b_off.reshape(NHEADS, NPTS, 2)
    bcat = jnp.concatenate([b_val.reshape(NHEADS, HDIM),
                            float(W) * bo2[..., 0], float(H) * bo2[..., 1],
                            b_attn.reshape(NHEADS, NPTS)], axis=1)[:, None, :]
    rp = reference_points.reshape(N, L, 2) * jnp.array([W, H], jnp.float32) - 0.5
    Wob = W_out.reshape(NHEADS, HDIM, E)
    bob = b_out.reshape(1, E)

    out = pl.pallas_call(
        _fused_kernel,
        grid=(N, NHEADS),
        in_specs=[
            pl.BlockSpec((1, L, E), lambda n, h: (n, 0, 0)),
            pl.BlockSpec((1, L, 2), lambda n, h: (n, 0, 0)),
            pl.BlockSpec((1, E, HDIM + 3 * NPTS), lambda n, h: (h, 0, 0)),
            pl.BlockSpec((1, 1, HDIM + 3 * NPTS), lambda n, h: (h, 0, 0)),
            pl.BlockSpec((1, HDIM, E), lambda n, h: (h, 0, 0)),
            pl.BlockSpec((1, E), lambda n, h: (0, 0)),
        ],
        out_specs=pl.BlockSpec((1, L, E), lambda n, h: (n, 0, 0)),
        out_shape=jax.ShapeDtypeStruct((N, L, E), jnp.float32),
        compiler_params=pltpu.CompilerParams(
            dimension_semantics=("parallel", "arbitrary")),
    )(qf, rp, Wcat, bcat, Wob, bob)
    return out.reshape(N, H, W, E)
